# baseline (speedup 1.0000x reference)
import functools

import jax
import jax.numpy as jnp
from jax import lax
from jax.experimental import pallas as pl
from jax.experimental.pallas import tpu as pltpu
from jax.experimental.pallas import tpu_sc as plsc

N = 10000
E = 320000
D_IN = 128
H = 128
D_OUT = 64
LAYERS = 3

NC = 2
NS = 16
NW = NC * NS
EPW = E // NW
EPT = 10240
CH = 128
NCHUNK = EPT // CH
NPAD = 10240
RPT = NPAD // NS

_f32 = jnp.float32


def _sc_agg_body(h_hbm, src_hbm, dst_hbm, agg_hbm,
                 acc_sh, didx_all, sidx_a, sidx_b, rows_a, rows_b,
                 lsem_a, lsem_b, gsem_a, gsem_b, ssem_a, ssem_b):
    c = lax.axis_index("c")
    s = lax.axis_index("s")
    wid = s * NC + c
    my_row0 = pl.multiple_of(s * RPT, 8)
    ebase = pl.multiple_of(wid * EPT, 8)

    zeros16 = jnp.zeros((16,), _f32)

    def _zrow(i, carry):
        for j in range(H // 16):
            rows_a[i, pl.ds(j * 16, 16)] = zeros16
        return carry
    lax.fori_loop(0, CH, _zrow, 0)
    for k in range(RPT // CH):
        off = pl.multiple_of(s * RPT + k * CH, 8)
        pltpu.sync_copy(rows_a, acc_sh.at[pl.ds(off, CH)])

    pltpu.sync_copy(dst_hbm.at[wid], didx_all)

    plsc.subcore_barrier()

    def _l_start(ch, sidx, sem):
        off = pl.multiple_of(ebase + ch * CH, 8)
        pltpu.async_copy(src_hbm.at[pl.ds(off, CH)], sidx, sem)

    def _l_wait(sidx, sem):
        pltpu.make_async_copy(src_hbm.at[pl.ds(0, CH)], sidx, sem).wait()

    def _g_start(sidx, rows, sem):
        pltpu.async_copy(h_hbm.at[sidx], rows, sem)

    def _g_wait(rows, sem):
        pltpu.make_async_copy(h_hbm.at[sidx_a], rows, sem).wait()

    def _s_start(ch, rows, sem):
        pltpu.async_copy(rows, acc_sh.at[didx_all.at[ch]], sem, add=True)

    def _s_wait(rows, sem):
        pltpu.make_async_copy(rows, acc_sh.at[didx_all.at[0]], sem).wait()

    _l_start(0, sidx_a, lsem_a)
    _l_start(1, sidx_b, lsem_b)
    _l_wait(sidx_a, lsem_a)
    _g_start(sidx_a, rows_a, gsem_a)

    _g_wait(rows_a, gsem_a)
    _s_start(0, rows_a, ssem_a)
    _l_start(2, sidx_a, lsem_a)
    _l_wait(sidx_b, lsem_b)
    _g_start(sidx_b, rows_b, gsem_b)

    def _half(ch, slot):
        if slot == 0:
            sidx_p, rows_p, lsem_p, gsem_p, ssem_p = (
                sidx_a, rows_a, lsem_a, gsem_a, ssem_a)
            sidx_q, rows_q, lsem_q, gsem_q, ssem_q = (
                sidx_b, rows_b, lsem_b, gsem_b, ssem_b)
        else:
            sidx_p, rows_p, lsem_p, gsem_p, ssem_p = (
                sidx_b, rows_b, lsem_b, gsem_b, ssem_b)
            sidx_q, rows_q, lsem_q, gsem_q, ssem_q = (
                sidx_a, rows_a, lsem_a, gsem_a, ssem_a)
        _g_wait(rows_p, gsem_p)
        _s_start(ch, rows_p, ssem_p)
        _l_start(jnp.minimum(ch + 2, NCHUNK - 1), sidx_p, lsem_p)
        _s_wait(rows_q, ssem_q)
        _l_wait(sidx_q, lsem_q)
        _g_start(sidx_q, rows_q, gsem_q)

    def _round(i, carry):
        _half(2 * i + 1, 1)
        _half(2 * i + 2, 0)
        return carry
    lax.fori_loop(0, (NCHUNK - 2) // 2, _round, 0)

    _g_wait(rows_b, gsem_b)
    _s_start(NCHUNK - 1, rows_b, ssem_b)
    _s_wait(rows_a, ssem_a)
    _s_wait(rows_b, ssem_b)
    _l_wait(sidx_a, lsem_a)

    plsc.subcore_barrier()

    row0 = pl.multiple_of(c * NPAD + s * RPT, 8)
    pltpu.sync_copy(acc_sh.at[pl.ds(my_row0, RPT)],
                    agg_hbm.at[pl.ds(row0, RPT)])


def _sc_deg_body(dst_hbm, deg_hbm, dacc_sh, didx_all, ones_v, sem):
    c = lax.axis_index("c")
    s = lax.axis_index("s")
    wid = s * NC + c
    my_row0 = pl.multiple_of(s * RPT, 8)

    zeros16 = jnp.zeros((16,), _f32)
    e0 = jnp.where(lax.iota(jnp.int32, 16) == 0, 1.0, 0.0).astype(_f32)

    def _drow(i, carry):
        for j in range(H // 16):
            ones_v[i, pl.ds(j * 16, 16)] = zeros16
        return carry
    lax.fori_loop(0, CH, _drow, 0)
    for k in range(RPT // CH):
        off = pl.multiple_of(s * RPT + k * CH, 8)
        pltpu.sync_copy(ones_v, dacc_sh.at[pl.ds(off, CH)])

    def _orow(i, carry):
        ones_v[i, pl.ds(0, 16)] = e0
        return carry
    lax.fori_loop(0, CH, _orow, 0)

    pltpu.sync_copy(dst_hbm.at[wid], didx_all)

    plsc.subcore_barrier()

    def _dchunk(r, carry):
        for b in range(5):
            pltpu.async_copy(ones_v, dacc_sh.at[didx_all.at[5 * r + b]],
                             sem, add=True)
        for b in range(5):
            pltpu.make_async_copy(ones_v, dacc_sh.at[didx_all.at[0]],
                                  sem).wait()
        return carry
    lax.fori_loop(0, NCHUNK // 5, _dchunk, 0)

    plsc.subcore_barrier()

    row0 = pl.multiple_of(c * NPAD + s * RPT, 8)
    pltpu.sync_copy(dacc_sh.at[pl.ds(my_row0, RPT)],
                    deg_hbm.at[pl.ds(row0, RPT)])


_SC_MESH = plsc.VectorSubcoreMesh(
    core_axis_name="c", subcore_axis_name="s", num_cores=NC, num_subcores=NS)

_sc_deg = functools.partial(
    pl.kernel,
    out_type=jax.ShapeDtypeStruct((2 * NPAD, H), _f32),
    mesh=_SC_MESH,
    scratch_types=[
        pltpu.VMEM_SHARED((NPAD, H), _f32),
        pltpu.VMEM((NCHUNK, CH), jnp.int32),
        pltpu.VMEM((CH, H), _f32),
        pltpu.SemaphoreType.DMA,
    ],
)(_sc_deg_body)

_sc_agg = functools.partial(
    pl.kernel,
    out_type=jax.ShapeDtypeStruct((2 * NPAD, H), _f32),
    mesh=_SC_MESH,
    scratch_types=[
        pltpu.VMEM_SHARED((NPAD, H), _f32),
        pltpu.VMEM((NCHUNK, CH), jnp.int32),
        pltpu.VMEM((CH,), jnp.int32),
        pltpu.VMEM((CH,), jnp.int32),
        pltpu.VMEM((CH, H), _f32),
        pltpu.VMEM((CH, H), _f32),
        pltpu.SemaphoreType.DMA,
        pltpu.SemaphoreType.DMA,
        pltpu.SemaphoreType.DMA,
        pltpu.SemaphoreType.DMA,
        pltpu.SemaphoreType.DMA,
        pltpu.SemaphoreType.DMA,
    ],
)(_sc_agg_body)


_R = 2000


def _inproj_body(x_ref, w_ref, b_ref, o_ref):
    o_ref[...] = jnp.dot(x_ref[...], w_ref[...],
                         preferred_element_type=_f32) + b_ref[...]


def _inproj(x, w, b):
    return pl.pallas_call(
        _inproj_body,
        grid=(N // _R,),
        in_specs=[pl.BlockSpec((_R, D_IN), lambda i: (i, 0)),
                  pl.BlockSpec((D_IN, H), lambda i: (0, 0)),
                  pl.BlockSpec((1, H), lambda i: (0, 0))],
        out_specs=pl.BlockSpec((_R, H), lambda i: (i, 0)),
        out_shape=jax.ShapeDtypeStruct((N, H), _f32),
    )(x, w, b.reshape(1, H))


def _den_body(d_ref, o_ref):
    deg = jnp.sum(d_ref[...], axis=(0, 2))
    o_ref[...] = (1.0 / (deg + 1.0))[:, None]


def _den(d):
    return pl.pallas_call(
        _den_body,
        grid=(N // _R,),
        in_specs=[pl.BlockSpec((2, _R, H), lambda i: (0, i, 0))],
        out_specs=pl.BlockSpec((_R, 1), lambda i: (i, 0)),
        out_shape=jax.ShapeDtypeStruct((N, 1), _f32),
    )(d)


def _update_body(last, p_ref, d_ref, h_ref, w_ref, b_ref, g_ref, be_ref,
                 wo_ref, bo_ref, o_ref):
    agg = p_ref[0] + p_ref[1] + h_ref[...]
    sc = agg * d_ref[...]
    z = jnp.dot(sc, w_ref[...], preferred_element_type=_f32) + b_ref[...]
    z = jnp.maximum(z, 0.0)
    mu = jnp.mean(z, axis=-1, keepdims=True)
    zc = z - mu
    var = jnp.mean(zc * zc, axis=-1, keepdims=True)
    y = zc * lax.rsqrt(var + 1e-5) * g_ref[...] + be_ref[...]
    if last:
        y = jnp.dot(y, wo_ref[...], preferred_element_type=_f32) + bo_ref[...]
    o_ref[...] = y


def _update(p, d, h, w, b, g, be, wo, bo, last):
    dout = D_OUT if last else H
    return pl.pallas_call(
        functools.partial(_update_body, last),
        grid=(N // _R,),
        in_specs=[pl.BlockSpec((2, _R, H), lambda i: (0, i, 0)),
                  pl.BlockSpec((_R, 1), lambda i: (i, 0)),
                  pl.BlockSpec((_R, H), lambda i: (i, 0)),
                  pl.BlockSpec((H, H), lambda i: (0, 0)),
                  pl.BlockSpec((1, H), lambda i: (0, 0)),
                  pl.BlockSpec((1, H), lambda i: (0, 0)),
                  pl.BlockSpec((1, H), lambda i: (0, 0)),
                  pl.BlockSpec((H, D_OUT), lambda i: (0, 0)),
                  pl.BlockSpec((1, D_OUT), lambda i: (0, 0))],
        out_specs=pl.BlockSpec((_R, dout), lambda i: (i, 0)),
        out_shape=jax.ShapeDtypeStruct((N, dout), _f32),
    )(p, d, h, w, b.reshape(1, H), g.reshape(1, H), be.reshape(1, H),
      wo, bo.reshape(1, D_OUT))


def kernel(nodes, edge_index, W_in, b_in, W_neigh, b_neigh, ln_g, ln_b,
           W_out, b_out):
    npd = EPT - EPW
    src = jnp.pad(edge_index[0].reshape(NW, EPW),
                  ((0, 0), (0, npd))).reshape(NW * EPT)
    pad_dst = jnp.broadcast_to(N + (jnp.arange(npd, dtype=jnp.int32)
                                    % (NPAD - N)), (NW, npd))
    dst = jnp.concatenate(
        [edge_index[1].reshape(NW, EPW), pad_dst], axis=1,
    ).reshape(NW, NCHUNK, CH)

    h = _inproj(nodes, W_in, b_in)
    deg = _sc_deg(dst)
    d = _den(deg.reshape(2, NPAD, H))

    agg = _sc_agg(h, src, dst)
    p = agg.reshape(2, NPAD, H)
    h = _update(p, d, h, W_neigh[0], b_neigh[0], ln_g[0], ln_b[0],
                W_out, b_out, last=False)

    for l in range(1, LAYERS):
        agg = _sc_agg(h, src, dst)
        p = agg.reshape(2, NPAD, H)
        h = _update(p, d, h, W_neigh[l], b_neigh[l], ln_g[l], ln_b[l],
                    W_out, b_out, last=(l == LAYERS - 1))

    return h


# speedup vs baseline: 2.2776x; 2.2776x over previous
import functools

import jax
import jax.numpy as jnp
from jax import lax
from jax.experimental import pallas as pl
from jax.experimental.pallas import tpu as pltpu
from jax.experimental.pallas import tpu_sc as plsc

N = 10000
E = 320000
D_IN = 128
H = 128
D_OUT = 64
LAYERS = 3

NC = 2
NS = 16
NW = NC * NS
EPW = E // NW
CH = 80
NCHUNK = EPW // CH
NPAD = 10240
RPT = NPAD // NS
ZR = 80

_f32 = jnp.float32


def _sc_agg_body(h_hbm, src_hbm, dst_hbm, agg_hbm,
                 acc_sh, sidx_v, didx_all, rows0, rows1,
                 gsem0, gsem1, ssem0, ssem1):
    c = lax.axis_index("c")
    s = lax.axis_index("s")
    wid = s * NC + c
    my_row0 = pl.multiple_of(s * RPT, 8)

    zeros16 = jnp.zeros((16,), _f32)

    def _zrow(i, carry):
        for j in range(H // 16):
            rows0[i, pl.ds(j * 16, 16)] = zeros16
        return carry
    lax.fori_loop(0, ZR, _zrow, 0)
    for k in range(RPT // ZR):
        off = pl.multiple_of(s * RPT + k * ZR, 8)
        pltpu.sync_copy(rows0, acc_sh.at[pl.ds(off, ZR)])

    pltpu.sync_copy(src_hbm.at[pl.ds(wid * EPW, EPW)], sidx_v)
    pltpu.sync_copy(dst_hbm.at[wid], didx_all)

    plsc.subcore_barrier()

    def _sidx(ch):
        return sidx_v.at[pl.ds(pl.multiple_of(ch * CH, 8), CH)]

    def _g_start(ch, rows, sem):
        pltpu.async_copy(h_hbm.at[_sidx(ch)], rows, sem)

    def _g_wait(rows, sem):
        pltpu.make_async_copy(h_hbm.at[_sidx(0)], rows, sem).wait()

    def _s_start(ch, rows, sem):
        pltpu.async_copy(rows, acc_sh.at[didx_all.at[ch]], sem, add=True)

    def _s_wait(rows, sem):
        pltpu.make_async_copy(rows, acc_sh.at[didx_all.at[0]], sem).wait()

    _g_start(0, rows0, gsem0)
    _g_wait(rows0, gsem0)
    _s_start(0, rows0, ssem0)
    _g_start(1, rows1, gsem1)

    def _round(gr, carry):
        r1 = 2 * gr + 1
        _g_wait(rows1, gsem1)
        _s_start(r1, rows1, ssem1)
        _s_wait(rows0, ssem0)
        _g_start(r1 + 1, rows0, gsem0)
        _g_wait(rows0, gsem0)
        _s_start(r1 + 1, rows0, ssem0)
        _s_wait(rows1, ssem1)
        _g_start(r1 + 2, rows1, gsem1)
        return carry
    lax.fori_loop(0, (NCHUNK - 3) // 2, _round, 0)

    last = NCHUNK - 1
    _g_wait(rows1, gsem1)
    _s_start(last - 1, rows1, ssem1)
    _s_wait(rows0, ssem0)
    _g_start(last, rows0, gsem0)
    _g_wait(rows0, gsem0)
    _s_start(last, rows0, ssem0)
    _s_wait(rows1, ssem1)
    _s_wait(rows0, ssem0)

    plsc.subcore_barrier()

    row0 = pl.multiple_of(c * NPAD + s * RPT, 8)
    pltpu.sync_copy(acc_sh.at[pl.ds(my_row0, RPT)],
                    agg_hbm.at[pl.ds(row0, RPT)])


def _sc_deg_body(dst_hbm, deg_hbm, dacc_sh, didx_all, ones_v, sem):
    c = lax.axis_index("c")
    s = lax.axis_index("s")
    wid = s * NC + c
    my_row0 = pl.multiple_of(s * RPT, 8)

    zeros16 = jnp.zeros((16,), _f32)
    e0 = jnp.where(lax.iota(jnp.int32, 16) == 0, 1.0, 0.0).astype(_f32)

    def _drow(i, carry):
        for j in range(H // 16):
            ones_v[i, pl.ds(j * 16, 16)] = zeros16
        return carry
    lax.fori_loop(0, CH, _drow, 0)
    for k in range(RPT // CH):
        off = pl.multiple_of(s * RPT + k * CH, 8)
        pltpu.sync_copy(ones_v, dacc_sh.at[pl.ds(off, CH)])

    def _orow(i, carry):
        ones_v[i, pl.ds(0, 16)] = e0
        return carry
    lax.fori_loop(0, CH, _orow, 0)

    pltpu.sync_copy(dst_hbm.at[wid], didx_all)

    plsc.subcore_barrier()

    def _dchunk(r, carry):
        for b in range(5):
            pltpu.async_copy(ones_v, dacc_sh.at[didx_all.at[5 * r + b]],
                             sem, add=True)
        for b in range(5):
            pltpu.make_async_copy(ones_v, dacc_sh.at[didx_all.at[0]],
                                  sem).wait()
        return carry
    lax.fori_loop(0, NCHUNK // 5, _dchunk, 0)

    plsc.subcore_barrier()

    row0 = pl.multiple_of(c * NPAD + s * RPT, 8)
    pltpu.sync_copy(dacc_sh.at[pl.ds(my_row0, RPT)],
                    deg_hbm.at[pl.ds(row0, RPT)])


_SC_MESH = plsc.VectorSubcoreMesh(
    core_axis_name="c", subcore_axis_name="s", num_cores=NC, num_subcores=NS)

_sc_deg = functools.partial(
    pl.kernel,
    out_type=jax.ShapeDtypeStruct((2 * NPAD, H), _f32),
    mesh=_SC_MESH,
    scratch_types=[
        pltpu.VMEM_SHARED((NPAD, H), _f32),
        pltpu.VMEM((NCHUNK, CH), jnp.int32),
        pltpu.VMEM((CH, H), _f32),
        pltpu.SemaphoreType.DMA,
    ],
)(_sc_deg_body)

_sc_agg = functools.partial(
    pl.kernel,
    out_type=jax.ShapeDtypeStruct((2 * NPAD, H), _f32),
    mesh=_SC_MESH,
    scratch_types=[
        pltpu.VMEM_SHARED((NPAD, H), _f32),
        pltpu.VMEM((EPW,), jnp.int32),
        pltpu.VMEM((NCHUNK, CH), jnp.int32),
        pltpu.VMEM((CH, H), _f32),
        pltpu.VMEM((CH, H), _f32),
        pltpu.SemaphoreType.DMA,
        pltpu.SemaphoreType.DMA,
        pltpu.SemaphoreType.DMA,
        pltpu.SemaphoreType.DMA,
    ],
)(_sc_agg_body)


_R = 2000


def _inproj_body(x_ref, w_ref, b_ref, o_ref):
    o_ref[...] = jnp.dot(x_ref[...], w_ref[...],
                         preferred_element_type=_f32) + b_ref[...]


def _inproj(x, w, b):
    return pl.pallas_call(
        _inproj_body,
        grid=(N // _R,),
        in_specs=[pl.BlockSpec((_R, D_IN), lambda i: (i, 0)),
                  pl.BlockSpec((D_IN, H), lambda i: (0, 0)),
                  pl.BlockSpec((1, H), lambda i: (0, 0))],
        out_specs=pl.BlockSpec((_R, H), lambda i: (i, 0)),
        out_shape=jax.ShapeDtypeStruct((N, H), _f32),
    )(x, w, b.reshape(1, H))


def _den_body(d_ref, o_ref):
    deg = jnp.sum(d_ref[...], axis=(0, 2))
    o_ref[...] = (1.0 / (deg + 1.0))[:, None]


def _den(d):
    return pl.pallas_call(
        _den_body,
        grid=(N // _R,),
        in_specs=[pl.BlockSpec((2, _R, H), lambda i: (0, i, 0))],
        out_specs=pl.BlockSpec((_R, 1), lambda i: (i, 0)),
        out_shape=jax.ShapeDtypeStruct((N, 1), _f32),
    )(d)


def _update_body(last, p_ref, d_ref, h_ref, w_ref, b_ref, g_ref, be_ref,
                 wo_ref, bo_ref, o_ref):
    agg = p_ref[0] + p_ref[1] + h_ref[...]
    sc = agg * d_ref[...]
    z = jnp.dot(sc, w_ref[...], preferred_element_type=_f32) + b_ref[...]
    z = jnp.maximum(z, 0.0)
    mu = jnp.mean(z, axis=-1, keepdims=True)
    zc = z - mu
    var = jnp.mean(zc * zc, axis=-1, keepdims=True)
    y = zc * lax.rsqrt(var + 1e-5) * g_ref[...] + be_ref[...]
    if last:
        y = jnp.dot(y, wo_ref[...], preferred_element_type=_f32) + bo_ref[...]
    o_ref[...] = y


def _update(p, d, h, w, b, g, be, wo, bo, last):
    dout = D_OUT if last else H
    return pl.pallas_call(
        functools.partial(_update_body, last),
        grid=(N // _R,),
        in_specs=[pl.BlockSpec((2, _R, H), lambda i: (0, i, 0)),
                  pl.BlockSpec((_R, 1), lambda i: (i, 0)),
                  pl.BlockSpec((_R, H), lambda i: (i, 0)),
                  pl.BlockSpec((H, H), lambda i: (0, 0)),
                  pl.BlockSpec((1, H), lambda i: (0, 0)),
                  pl.BlockSpec((1, H), lambda i: (0, 0)),
                  pl.BlockSpec((1, H), lambda i: (0, 0)),
                  pl.BlockSpec((H, D_OUT), lambda i: (0, 0)),
                  pl.BlockSpec((1, D_OUT), lambda i: (0, 0))],
        out_specs=pl.BlockSpec((_R, dout), lambda i: (i, 0)),
        out_shape=jax.ShapeDtypeStruct((N, dout), _f32),
    )(p, d, h, w, b.reshape(1, H), g.reshape(1, H), be.reshape(1, H),
      wo, bo.reshape(1, D_OUT))


def kernel(nodes, edge_index, W_in, b_in, W_neigh, b_neigh, ln_g, ln_b,
           W_out, b_out):
    src = edge_index[0]
    dst = edge_index[1].reshape(NW, NCHUNK, CH)

    h = _inproj(nodes, W_in, b_in)
    deg = _sc_deg(dst)
    d = _den(deg.reshape(2, NPAD, H))

    agg = _sc_agg(h, src, dst)
    p = agg.reshape(2, NPAD, H)
    h = _update(p, d, h, W_neigh[0], b_neigh[0], ln_g[0], ln_b[0],
                W_out, b_out, last=False)

    for l in range(1, LAYERS):
        agg = _sc_agg(h, src, dst)
        p = agg.reshape(2, NPAD, H)
        h = _update(p, d, h, W_neigh[l], b_neigh[l], ln_g[l], ln_b[l],
                    W_out, b_out, last=(l == LAYERS - 1))

    return h
